# baseline (speedup 1.0000x reference)
import functools

import jax
import jax.numpy as jnp
from jax import lax
from jax.experimental import pallas as pl
from jax.experimental.pallas import tpu as pltpu
from jax.experimental.pallas import tpu_sc as plsc

N, C, D, GH, GW = 4, 12, 8, 16, 16
H = W = 512
NWORK = 32
ROWS_PER_W = (N * H) // NWORK
CHUNK = 8
WPC = 7
NW = C // 2
ZS = WPC
YS = D * WPC
RT = GH * YS
TBL = GW * RT
NGRP = W // 16
_ILV = plsc.PackFormat.INTERLEAVED


def _sc_body(t2_hbm, guide_hbm, y0o_hbm, y1o_hbm, wy_hbm, omy_hbm, out_hbm,
             t2_v, rt_v, gd_v, out_v, y0o_v, y1o_v, wy_v, omy_v,
             osem0, osem1, gsem0, gsem1):
    wid = lax.axis_index("c") * 16 + lax.axis_index("s")
    n = wid // 8
    iblk = (wid % 8) * ROWS_PER_W

    pltpu.sync_copy(t2_hbm.at[n], t2_v)
    pltpu.sync_copy(y0o_hbm, y0o_v)
    pltpu.sync_copy(y1o_hbm, y1o_v)
    pltpu.sync_copy(wy_hbm, wy_v)
    pltpu.sync_copy(omy_hbm, omy_v)

    pltpu.make_async_copy(
        guide_hbm.at[n, pl.ds(iblk, CHUNK), :], gd_v.at[0], gsem0).start()
    pltpu.make_async_copy(
        guide_hbm.at[n, pl.ds(iblk + CHUNK, CHUNK), :], gd_v.at[1],
        gsem1).start()

    def chunk_pair_body(cp, carry):
      for b, osem, gsem in ((0, osem0, gsem0), (1, osem1, gsem1)):
        ch = cp * 2 + b
        i0 = iblk + ch * CHUNK
        pltpu.make_async_copy(
            guide_hbm.at[n, pl.ds(i0, CHUNK), :], gd_v.at[b], gsem).wait()

        @pl.when(ch >= 2)
        def _():
            pltpu.make_async_copy(
                out_v.at[b], out_hbm.at[n, :, pl.ds(i0, CHUNK), :], osem
            ).wait()

        def row_body(r, carry):
            i = i0 + r
            i15 = i * (GW - 1)
            x0i = i15 // (H - 1)
            wx = i15.astype(jnp.float32) * (1.0 / (H - 1)) - x0i.astype(jnp.float32)
            x1i = jnp.minimum(x0i + 1, GW - 1)
            x0 = x0i * RT
            x1 = x1i * RT

            wxp = plsc.pack(jnp.full((16,), wx, jnp.float32),
                            jnp.full((16,), wx, jnp.float32), format=_ILV)

            def rt_body(k, carry):
                offs = [k * 128 + q * 16 for q in range(8)]
                v0s = [plsc.bitcast(t2_v[pl.ds(x0 + o, 16)], jnp.bfloat16)
                       for o in offs]
                v1s = [plsc.bitcast(t2_v[pl.ds(x1 + o, 16)], jnp.bfloat16)
                       for o in offs]
                for o, a, b in zip(offs, v0s, v1s):
                    rt_v[pl.ds(o, 16)] = plsc.bitcast(a + wxp * (b - a),
                                                      jnp.int32)
                return carry

            lax.fori_loop(0, RT // 128, rt_body, 0)

            def grp_body(gj, carry):
                j0 = gj * 16
                g = gd_v[b, r, pl.ds(j0, 16)]
                iz = jnp.clip((g + 1.0) * (0.5 * (D - 1)), 0.0, float(D - 1))
                z0 = iz.astype(jnp.int32)
                wz = iz - z0.astype(jnp.float32)
                z1 = jnp.minimum(z0 + 1, D - 1)
                y0o = y0o_v[pl.ds(j0, 16)]
                y1o = y1o_v[pl.ds(j0, 16)]
                wy = wy_v[pl.ds(j0, 16)]
                omy = omy_v[pl.ds(j0, 16)]
                z0o = z0 * ZS
                z1o = z1 * ZS
                b00 = z0o + y0o
                b01 = z0o + y1o
                b10 = z1o + y0o
                b11 = z1o + y1o
                omz = 1.0 - wz
                pw00 = plsc.pack(omz * omy, omz * omy, format=_ILV)
                pw01 = plsc.pack(omz * wy, omz * wy, format=_ILV)
                pw10 = plsc.pack(wz * omy, wz * omy, format=_ILV)
                pw11 = plsc.pack(wz * wy, wz * wy, format=_ILV)
                g = [[plsc.bitcast(plsc.load_gather(rt_v, [bb + w]),
                                   jnp.bfloat16)
                      for bb in (b00, b01, b10, b11)]
                     for w in range(NW)]
                for w in range(NW):
                    g0, g1, g2, g3 = g[w]
                    v = (g0 * pw00 + g1 * pw01) + (g2 * pw10 + g3 * pw11)
                    va, vb = plsc.unpack(v, format=_ILV)
                    out_v[b, 2 * w, r, pl.ds(j0, 16)] = va
                    out_v[b, 2 * w + 1, r, pl.ds(j0, 16)] = vb
                return carry

            lax.fori_loop(0, NGRP, grp_body, 0, unroll=4)
            return carry

        lax.fori_loop(0, CHUNK, row_body, 0)
        pltpu.make_async_copy(
            out_v.at[b], out_hbm.at[n, :, pl.ds(i0, CHUNK), :], osem
        ).start()

        @pl.when(ch + 2 < ROWS_PER_W // CHUNK)
        def _():
            pltpu.make_async_copy(
                guide_hbm.at[n, pl.ds(i0 + 2 * CHUNK, CHUNK), :],
                gd_v.at[b], gsem).start()
      return carry

    lax.fori_loop(0, ROWS_PER_W // (2 * CHUNK), chunk_pair_body, 0)
    for b, osem in ((0, osem0), (1, osem1)):
        pltpu.make_async_copy(
            out_v.at[b], out_hbm.at[n, :, pl.ds(iblk, CHUNK), :], osem
        ).wait()


@jax.jit
def kernel(bilateral_grid, guidemap):
    t2 = jnp.transpose(bilateral_grid, (0, 4, 3, 2, 1))
    t2 = jnp.pad(t2, ((0, 0), (0, 0), (0, 0), (0, 0), (0, 2 * WPC - C)))
    t2 = t2.astype(jnp.bfloat16).reshape(N, GW, GH, D, WPC, 2)
    t2 = jax.lax.bitcast_convert_type(t2, jnp.int32).reshape(N, TBL)

    t = (jnp.arange(512, dtype=jnp.float32) / (H - 1)) * 2.0 - 1.0
    pos = jnp.clip((t + 1.0) * 0.5 * (GW - 1), 0.0, float(GW - 1))
    f0 = jnp.floor(pos)
    idx0 = f0.astype(jnp.int32)
    idx1 = jnp.minimum(idx0 + 1, GW - 1)
    frac = pos - f0

    mesh = plsc.VectorSubcoreMesh(core_axis_name="c", subcore_axis_name="s")
    run = functools.partial(
        pl.kernel,
        mesh=mesh,
        compiler_params=pltpu.CompilerParams(needs_layout_passes=False),
        out_type=jax.ShapeDtypeStruct((N, C, H, W), jnp.float32),
        scratch_types=[
            pltpu.VMEM((TBL,), jnp.int32),
            pltpu.VMEM((RT,), jnp.int32),
            pltpu.VMEM((2, CHUNK, W), jnp.float32),
            pltpu.VMEM((2, C, CHUNK, W), jnp.float32),
            pltpu.VMEM((512,), jnp.int32),
            pltpu.VMEM((512,), jnp.int32),
            pltpu.VMEM((512,), jnp.float32),
            pltpu.VMEM((512,), jnp.float32),
            pltpu.SemaphoreType.DMA,
            pltpu.SemaphoreType.DMA,
            pltpu.SemaphoreType.DMA,
            pltpu.SemaphoreType.DMA,
        ],
    )(_sc_body)
    return run(t2, guidemap, idx0 * YS, idx1 * YS, frac, 1.0 - frac)


# speedup vs baseline: 1.1532x; 1.1532x over previous
import functools

import jax
import jax.numpy as jnp
from jax import lax
from jax.experimental import pallas as pl
from jax.experimental.pallas import tpu as pltpu
from jax.experimental.pallas import tpu_sc as plsc

N, C, D, GH, GW = 4, 12, 8, 16, 16
H = W = 512
NWORK = 32
ROWS_PER_W = (N * H) // NWORK
CHUNK = 4
WPC = 7
NW = C // 2
ZS = WPC
YS = D * WPC
RT = GH * YS
TBL = GW * RT
NGRP = W // 16
_ILV = plsc.PackFormat.INTERLEAVED


def _sc_body(t2_hbm, guide_hbm, y0o_hbm, y1o_hbm, wy_hbm, omy_hbm, out_hbm,
             t2_v, rt_v, gd_v, out_v, y0o_v, y1o_v, wy_v, omy_v,
             osem0, osem1, gsem0, gsem1):
    wid = lax.axis_index("c") * 16 + lax.axis_index("s")
    n = wid // 8
    iblk = (wid % 8) * ROWS_PER_W

    pltpu.sync_copy(t2_hbm.at[n], t2_v)
    pltpu.sync_copy(y0o_hbm, y0o_v)
    pltpu.sync_copy(y1o_hbm, y1o_v)
    pltpu.sync_copy(wy_hbm, wy_v)
    pltpu.sync_copy(omy_hbm, omy_v)

    pltpu.make_async_copy(
        guide_hbm.at[n, pl.ds(iblk, CHUNK), :], gd_v.at[0], gsem0).start()
    pltpu.make_async_copy(
        guide_hbm.at[n, pl.ds(iblk + CHUNK, CHUNK), :], gd_v.at[1],
        gsem1).start()

    def chunk_pair_body(cp, carry):
      for b, osem, gsem in ((0, osem0, gsem0), (1, osem1, gsem1)):
        ch = cp * 2 + b
        i0 = iblk + ch * CHUNK
        pltpu.make_async_copy(
            guide_hbm.at[n, pl.ds(i0, CHUNK), :], gd_v.at[b], gsem).wait()

        @pl.when(ch >= 2)
        def _():
            pltpu.make_async_copy(
                out_v.at[b], out_hbm.at[n, :, pl.ds(i0, CHUNK), :], osem
            ).wait()

        def rt_row(r, carry):
            i = i0 + r
            i15 = i * (GW - 1)
            x0i = i15 // (H - 1)
            wx = i15.astype(jnp.float32) * (1.0 / (H - 1)) - x0i.astype(jnp.float32)
            x1i = jnp.minimum(x0i + 1, GW - 1)
            x0 = x0i * TBL // GW
            x1 = x1i * TBL // GW
            rbase = r * RT

            wxp = plsc.pack(jnp.full((16,), wx, jnp.float32),
                            jnp.full((16,), wx, jnp.float32), format=_ILV)

            def rt_body(k, carry):
                offs = [k * 128 + q * 16 for q in range(8)]
                v0s = [plsc.bitcast(t2_v[pl.ds(x0 + o, 16)], jnp.bfloat16)
                       for o in offs]
                v1s = [plsc.bitcast(t2_v[pl.ds(x1 + o, 16)], jnp.bfloat16)
                       for o in offs]
                for o, a, bb in zip(offs, v0s, v1s):
                    rt_v[pl.ds(rbase + o, 16)] = plsc.bitcast(
                        a + wxp * (bb - a), jnp.int32)
                return carry

            lax.fori_loop(0, RT // 128, rt_body, 0)
            return carry

        lax.fori_loop(0, CHUNK, rt_row, 0)

        def grp_body(gj, carry):
            j0 = gj * 16
            y0o = y0o_v[pl.ds(j0, 16)]
            y1o = y1o_v[pl.ds(j0, 16)]
            wy = wy_v[pl.ds(j0, 16)]
            omy = omy_v[pl.ds(j0, 16)]
            for r in range(CHUNK):
                g = gd_v[b, r, pl.ds(j0, 16)]
                iz = jnp.clip((g + 1.0) * (0.5 * (D - 1)), 0.0, float(D - 1))
                z0 = iz.astype(jnp.int32)
                wz = iz - z0.astype(jnp.float32)
                z1 = jnp.minimum(z0 + 1, D - 1)
                z0o = z0 * ZS
                z1o = z1 * ZS
                b00 = z0o + y0o
                b01 = z0o + y1o
                b10 = z1o + y0o
                b11 = z1o + y1o
                omz = 1.0 - wz
                pw00 = plsc.pack(omz * omy, omz * omy, format=_ILV)
                pw01 = plsc.pack(omz * wy, omz * wy, format=_ILV)
                pw10 = plsc.pack(wz * omy, wz * omy, format=_ILV)
                pw11 = plsc.pack(wz * wy, wz * wy, format=_ILV)
                gs = [[plsc.bitcast(
                           plsc.load_gather(rt_v, [bb + (r * RT + w)]),
                           jnp.bfloat16)
                       for bb in (b00, b01, b10, b11)]
                      for w in range(NW)]
                for w in range(NW):
                    g0, g1, g2, g3 = gs[w]
                    v = (g0 * pw00 + g1 * pw01) + (g2 * pw10 + g3 * pw11)
                    va, vb = plsc.unpack(v, format=_ILV)
                    out_v[b, 2 * w, r, pl.ds(j0, 16)] = va
                    out_v[b, 2 * w + 1, r, pl.ds(j0, 16)] = vb
            return carry

        lax.fori_loop(0, NGRP, grp_body, 0)
        pltpu.make_async_copy(
            out_v.at[b], out_hbm.at[n, :, pl.ds(i0, CHUNK), :], osem
        ).start()

        @pl.when(ch + 2 < ROWS_PER_W // CHUNK)
        def _():
            pltpu.make_async_copy(
                guide_hbm.at[n, pl.ds(i0 + 2 * CHUNK, CHUNK), :],
                gd_v.at[b], gsem).start()
      return carry

    lax.fori_loop(0, ROWS_PER_W // (2 * CHUNK), chunk_pair_body, 0)
    for b, osem in ((0, osem0), (1, osem1)):
        pltpu.make_async_copy(
            out_v.at[b], out_hbm.at[n, :, pl.ds(iblk, CHUNK), :], osem
        ).wait()


@jax.jit
def kernel(bilateral_grid, guidemap):
    t2 = jnp.transpose(bilateral_grid, (0, 4, 3, 2, 1))
    t2 = jnp.pad(t2, ((0, 0), (0, 0), (0, 0), (0, 0), (0, 2 * WPC - C)))
    t2 = t2.astype(jnp.bfloat16).reshape(N, GW, GH, D, WPC, 2)
    t2 = jax.lax.bitcast_convert_type(t2, jnp.int32).reshape(N, TBL)

    t = (jnp.arange(512, dtype=jnp.float32) / (H - 1)) * 2.0 - 1.0
    pos = jnp.clip((t + 1.0) * 0.5 * (GW - 1), 0.0, float(GW - 1))
    f0 = jnp.floor(pos)
    idx0 = f0.astype(jnp.int32)
    idx1 = jnp.minimum(idx0 + 1, GW - 1)
    frac = pos - f0

    mesh = plsc.VectorSubcoreMesh(core_axis_name="c", subcore_axis_name="s")
    run = functools.partial(
        pl.kernel,
        mesh=mesh,
        compiler_params=pltpu.CompilerParams(needs_layout_passes=False),
        out_type=jax.ShapeDtypeStruct((N, C, H, W), jnp.float32),
        scratch_types=[
            pltpu.VMEM((TBL,), jnp.int32),
            pltpu.VMEM((CHUNK * RT,), jnp.int32),
            pltpu.VMEM((2, CHUNK, W), jnp.float32),
            pltpu.VMEM((2, C, CHUNK, W), jnp.float32),
            pltpu.VMEM((512,), jnp.int32),
            pltpu.VMEM((512,), jnp.int32),
            pltpu.VMEM((512,), jnp.float32),
            pltpu.VMEM((512,), jnp.float32),
            pltpu.SemaphoreType.DMA,
            pltpu.SemaphoreType.DMA,
            pltpu.SemaphoreType.DMA,
            pltpu.SemaphoreType.DMA,
        ],
    )(_sc_body)
    return run(t2, guidemap, idx0 * YS, idx1 * YS, frac, 1.0 - frac)
